# Initial kernel scaffold; baseline (speedup 1.0000x reference)
import functools

import jax
import jax.numpy as jnp
from jax import lax
from jax.experimental import pallas as pl
from jax.experimental.pallas import tpu as pltpu
from jax.experimental.pallas import tpu_sc as plsc

f32 = jnp.float32

NN = 50000
NE = 800000
NODE_IN = 4
HID = 128
CC = 32
NCH = HID // CC

NC, NS = 2, 16
NNP = 51200
STRIPE = NNP // NS
B = 80
ROWS = NE // B
TROWS = ROWS // NS
CH = 5
CROWS = TROWS // CH
KB = 5
NB = CROWS // KB

BN = 2000
GN = NN // BN

_mesh = plsc.VectorSubcoreMesh(core_axis_name="c", subcore_axis_name="s")
_HI = lax.Precision.HIGHEST


def _dot(a, b):
    return lax.dot_general(a, b, (((1,), (0,)), ((), ())), precision=_HI,
                           preferred_element_type=f32)


def _deg_body(dst_r, zeros_h, deg_out, dst_v, ones_v, shared):
    w = lax.axis_index("s")
    cid = lax.axis_index("c")

    @pl.when(cid == 0)
    def _():
        for i in range(B // 16):
            ones_v[pl.ds(i * 16, 16)] = jnp.ones((16,), f32)
        pltpu.sync_copy(zeros_h, shared.at[pl.ds(w * STRIPE, STRIPE)])
        plsc.subcore_barrier()

        def chunk(c, carry):
            row0 = w * TROWS + c * CROWS
            pltpu.sync_copy(dst_r.at[pl.ds(row0, CROWS)], dst_v)

            def blk(j, carry2):
                pltpu.sync_copy(ones_v, shared.at[dst_v.at[j]], add=True)
                return carry2

            return lax.fori_loop(0, CROWS, blk, carry)

        lax.fori_loop(0, CH, chunk, 0)
        plsc.subcore_barrier()
        pltpu.sync_copy(shared.at[pl.ds(w * STRIPE, STRIPE)],
                        deg_out.at[pl.ds(w * STRIPE, STRIPE)])


_deg_call = pl.kernel(
    _deg_body,
    out_type=jax.ShapeDtypeStruct((NNP,), f32),
    mesh=_mesh,
    scratch_types=[
        pltpu.VMEM((CROWS, B), jnp.int32),
        pltpu.VMEM((B,), f32),
        pltpu.VMEM_SHARED((NNP,), f32),
    ],
)


def _run_agg_pass(tab, out, src_r, dst_r, zeros_h, src_v, dst_v, rows_v,
                  gsem, shared, w):
    pltpu.sync_copy(zeros_h, shared.at[pl.ds(w * STRIPE, STRIPE), :])
    plsc.subcore_barrier()

    def chunk(c, carry):
        row0 = w * TROWS + c * CROWS
        pltpu.sync_copy(src_r.at[pl.ds(row0, CROWS)], src_v)
        pltpu.sync_copy(dst_r.at[pl.ds(row0, CROWS)], dst_v)

        def batch(bb, carry2):
            r0 = bb * KB
            cps = [pltpu.async_copy(tab.at[src_v.at[r0 + k]], rows_v.at[k],
                                    gsem) for k in range(KB)]
            for cp in cps:
                cp.wait()
            for k in range(KB):
                pltpu.sync_copy(rows_v.at[k], shared.at[dst_v.at[r0 + k]],
                                add=True)
            return carry2

        return lax.fori_loop(0, NB, batch, carry)

    lax.fori_loop(0, CH, chunk, 0)
    plsc.subcore_barrier()
    pltpu.sync_copy(shared.at[pl.ds(w * STRIPE, STRIPE), :],
                    out.at[pl.ds(w * STRIPE, STRIPE), :])


def _agg4_body(src_r, dst_r, tab, zeros_h, out, src_v, dst_v, rows_v, gsem,
               shared):
    w = lax.axis_index("s")
    cid = lax.axis_index("c")

    @pl.when(cid == 0)
    def _():
        _run_agg_pass(tab, out, src_r, dst_r, zeros_h, src_v, dst_v, rows_v,
                      gsem, shared, w)


_agg4_call = pl.kernel(
    _agg4_body,
    out_type=jax.ShapeDtypeStruct((NNP, NODE_IN), f32),
    mesh=_mesh,
    scratch_types=[
        pltpu.VMEM((CROWS, B), jnp.int32),
        pltpu.VMEM((CROWS, B), jnp.int32),
        pltpu.VMEM((KB, B, NODE_IN), f32),
        pltpu.SemaphoreType.DMA,
        pltpu.VMEM_SHARED((NNP, NODE_IN), f32),
    ],
)


def _agg32_body(src_r, dst_r, t0, t1, t2, t3, zeros_h, o0, o1, o2, o3,
                src_v, dst_v, rows_v, gsem, shared):
    w = lax.axis_index("s")
    cid = lax.axis_index("c")
    tabs = (t0, t1, t2, t3)
    outs = (o0, o1, o2, o3)
    for c in range(NCH):
        @pl.when(cid == c // 2)
        def _(c=c):
            _run_agg_pass(tabs[c], outs[c], src_r, dst_r, zeros_h, src_v,
                          dst_v, rows_v, gsem, shared, w)


_agg32_call = pl.kernel(
    _agg32_body,
    out_type=tuple(jax.ShapeDtypeStruct((NNP, CC), f32) for _ in range(NCH)),
    mesh=_mesh,
    scratch_types=[
        pltpu.VMEM((CROWS, B), jnp.int32),
        pltpu.VMEM((CROWS, B), jnp.int32),
        pltpu.VMEM((KB, B, CC), f32),
        pltpu.SemaphoreType.DMA,
        pltpu.VMEM_SHARED((NNP, CC), f32),
    ],
)


def _prep_body(deg_ref, x_ref, dinv_ref, xt_ref):
    dinv = lax.rsqrt(deg_ref[...] + 1.0)
    dinv_ref[...] = dinv
    xt_ref[...] = x_ref[...] * dinv


_prep_call = pl.pallas_call(
    _prep_body,
    grid=(GN,),
    in_specs=[
        pl.BlockSpec((BN, 1), lambda i: (i, 0)),
        pl.BlockSpec((BN, NODE_IN), lambda i: (i, 0)),
    ],
    out_specs=[
        pl.BlockSpec((BN, 1), lambda i: (i, 0)),
        pl.BlockSpec((BN, NODE_IN), lambda i: (i, 0)),
    ],
    out_shape=[
        jax.ShapeDtypeStruct((NN, 1), f32),
        jax.ShapeDtypeStruct((NN, NODE_IN), f32),
    ],
)


def _l1_body(a_ref, x_ref, dinv_ref, w_ref, b_ref, o0, o1, o2, o3):
    dinv = dinv_ref[...]
    agg = dinv * (a_ref[...] + dinv * x_ref[...])
    h = jnp.maximum(_dot(agg, w_ref[...]) + b_ref[...], 0.0)
    ht = dinv * h
    for i, o in enumerate((o0, o1, o2, o3)):
        o[...] = ht[:, i * CC:(i + 1) * CC]


_l1_call = pl.pallas_call(
    _l1_body,
    grid=(GN,),
    in_specs=[
        pl.BlockSpec((BN, NODE_IN), lambda i: (i, 0)),
        pl.BlockSpec((BN, NODE_IN), lambda i: (i, 0)),
        pl.BlockSpec((BN, 1), lambda i: (i, 0)),
        pl.BlockSpec((NODE_IN, HID), lambda i: (0, 0)),
        pl.BlockSpec((1, HID), lambda i: (0, 0)),
    ],
    out_specs=[pl.BlockSpec((BN, CC), lambda i: (i, 0)) for _ in range(NCH)],
    out_shape=[jax.ShapeDtypeStruct((NN, CC), f32) for _ in range(NCH)],
)


def _l2_body(ag0, ag1, ag2, ag3, ht0, ht1, ht2, ht3, dinv_ref,
             w20, w21, w22, w23, b2, wp, bp, ang, wa1, ba1, wa2, ba2,
             wc1g, wc1a, bc1, wc2, bc2, out_ref, acc):
    i = pl.program_id(0)
    dinv = dinv_ref[...]
    ags = (ag0, ag1, ag2, ag3)
    hts = (ht0, ht1, ht2, ht3)
    w2s = (w20, w21, w22, w23)
    t = b2[...]
    for c in range(NCH):
        part = dinv * (ags[c][...] + hts[c][...])
        t = t + _dot(part, w2s[c][...])
    t = jnp.maximum(t, 0.0)
    psum = jnp.sum(t, axis=0, keepdims=True)

    @pl.when(i == 0)
    def _():
        acc[...] = psum

    @pl.when(i > 0)
    def _():
        acc[...] = acc[...] + psum

    @pl.when(i == GN - 1)
    def _():
        g = acc[...] * (1.0 / NN)
        gp = _dot(g, wp[...]) + bp[...]
        a = jnp.maximum(_dot(ang[...], wa1[...]) + ba1[...], 0.0)
        a = jnp.maximum(_dot(a, wa2[...]) + ba2[...], 0.0)
        o = jnp.maximum(_dot(gp, wc1g[...]) + _dot(a, wc1a[...]) + bc1[...],
                        0.0)
        out_ref[...] = _dot(o, wc2[...]) + bc2[...]


def _full(shape):
    return pl.BlockSpec(shape, lambda i: tuple(0 for _ in shape))


_l2_call = pl.pallas_call(
    _l2_body,
    grid=(GN,),
    in_specs=(
        [pl.BlockSpec((BN, CC), lambda i: (i, 0)) for _ in range(2 * NCH)]
        + [pl.BlockSpec((BN, 1), lambda i: (i, 0))]
        + [_full((CC, HID)) for _ in range(NCH)]
        + [_full((1, HID)), _full((HID, HID)), _full((1, HID)),
           _full((1, 12)), _full((12, 32)), _full((1, 32)),
           _full((32, 32)), _full((1, 32)),
           _full((HID, HID)), _full((32, HID)), _full((1, HID)),
           _full((HID, 10)), _full((1, 10))]
    ),
    out_specs=pl.BlockSpec((1, 10), lambda i: (0, 0)),
    out_shape=jax.ShapeDtypeStruct((1, 10), f32),
    scratch_shapes=[pltpu.VMEM((1, HID), f32)],
)


def kernel(x, edge_index, angles, W1, b1, W2, b2, Wp, bp, Wa1, ba1, Wa2, ba2,
           Wc1, bc1, Wc2, bc2):
    src_r = edge_index[0].astype(jnp.int32).reshape(ROWS, B)
    dst_r = edge_index[1].astype(jnp.int32).reshape(ROWS, B)
    zeros1 = jnp.zeros((STRIPE,), f32)
    zeros4 = jnp.zeros((STRIPE, NODE_IN), f32)
    zeros32 = jnp.zeros((STRIPE, CC), f32)

    deg_p = _deg_call(dst_r, zeros1)
    deg_col = deg_p[:NN].reshape(NN, 1)
    dinv, xt = _prep_call(deg_col, x)

    agg1 = _agg4_call(src_r, dst_r, xt, zeros4)[:NN]
    hts = _l1_call(agg1, x, dinv, W1, b1.reshape(1, HID))

    ags = _agg32_call(src_r, dst_r, *hts, zeros32)
    ags = [a[:NN] for a in ags]

    w2s = [W2[c * CC:(c + 1) * CC, :] for c in range(NCH)]
    out = _l2_call(
        *ags, *hts, dinv, *w2s, b2.reshape(1, HID),
        Wp, bp.reshape(1, HID), angles, Wa1, ba1.reshape(1, 32),
        Wa2, ba2.reshape(1, 32), Wc1[:HID, :], Wc1[HID:, :],
        bc1.reshape(1, HID), Wc2, bc2.reshape(1, 10),
    )
    return out


# speedup vs baseline: 12.9218x; 12.9218x over previous
import jax
import jax.numpy as jnp
from jax import lax
from jax.experimental import pallas as pl
from jax.experimental.pallas import tpu as pltpu
from jax.experimental.pallas import tpu_sc as plsc

f32 = jnp.float32

NN = 50000
NE = 800000
NODE_IN = 4
D1 = 16
HID = 128
CC = 16
NCH = HID // CC

NC, NS = 2, 16
NNP = 51200
STRIPE = NNP // NS
B = 640
ROWS = 1280
NE_P = ROWS * B
TROWS = ROWS // NS
CH = 5
CROWS = TROWS // CH
KB = 4
NB = CROWS // KB
HTROWS = TROWS // 2
HCROWS = HTROWS // CH

BN = 2000
GN = NN // BN

_mesh = plsc.VectorSubcoreMesh(core_axis_name="c", subcore_axis_name="s")
_sc_params = pltpu.CompilerParams(use_tc_tiling_on_sc=False)


def _dot(a, b):
    return lax.dot_general(a, b, (((1,), (0,)), ((), ())),
                           preferred_element_type=f32)


def _deg_body(dst_r, zeros_h, deg_out, dst_v, ones_v, shared):
    w = lax.axis_index("s")
    cid = lax.axis_index("c")
    for i in range(B // 16):
        ones_v[pl.ds(i * 16, 16)] = jnp.ones((16,), f32)
    pltpu.sync_copy(zeros_h, shared.at[pl.ds(w * STRIPE, STRIPE)])
    plsc.subcore_barrier()

    def chunk(c, carry):
        row0 = (cid * NS + w) * HTROWS + c * HCROWS
        pltpu.sync_copy(dst_r.at[pl.ds(row0, HCROWS)], dst_v)

        def blk(j, carry2):
            pltpu.sync_copy(ones_v, shared.at[dst_v.at[j]], add=True)
            return carry2

        return lax.fori_loop(0, HCROWS, blk, carry)

    lax.fori_loop(0, CH, chunk, 0)
    plsc.subcore_barrier()
    out = deg_out.at[cid]
    pltpu.sync_copy(shared.at[pl.ds(w * STRIPE, STRIPE)],
                    out.at[pl.ds(w * STRIPE, STRIPE)])


_deg_call = pl.kernel(
    _deg_body,
    out_type=jax.ShapeDtypeStruct((NC, NNP), f32),
    mesh=_mesh,
    compiler_params=_sc_params,
    scratch_types=[
        pltpu.VMEM((HCROWS, B), jnp.int32),
        pltpu.VMEM((B,), f32),
        pltpu.VMEM_SHARED((NNP,), f32),
    ],
)


def _run_agg_pass(tab, out, src_r, dst_r, zeros_h, src_v, dst_v, rows_v,
                  gsem, shared, w, row_base, crows):
    pltpu.sync_copy(zeros_h, shared.at[pl.ds(w * STRIPE, STRIPE), :])
    plsc.subcore_barrier()
    nb = crows // KB

    def chunk(c, carry):
        row0 = row_base + c * crows
        pltpu.sync_copy(src_r.at[pl.ds(row0, crows)], src_v)
        pltpu.sync_copy(dst_r.at[pl.ds(row0, crows)], dst_v)

        def batch(bb, carry2):
            r0 = bb * KB
            cps = [pltpu.async_copy(tab.at[src_v.at[r0 + k]], rows_v.at[k],
                                    gsem) for k in range(KB)]
            for cp in cps:
                cp.wait()
            for k in range(KB):
                pltpu.sync_copy(rows_v.at[k], shared.at[dst_v.at[r0 + k]],
                                add=True)
            return carry2

        return lax.fori_loop(0, nb, batch, carry)

    lax.fori_loop(0, CH, chunk, 0)
    plsc.subcore_barrier()
    pltpu.sync_copy(shared.at[pl.ds(w * STRIPE, STRIPE), :],
                    out.at[pl.ds(w * STRIPE, STRIPE), :])


def _agg16_body(src_r, dst_r, tab, zeros_h, out, src_v, dst_v, rows_v, gsem,
                shared):
    w = lax.axis_index("s")
    cid = lax.axis_index("c")
    wh = cid * NS + w
    _run_agg_pass(tab, out.at[cid], src_r, dst_r, zeros_h, src_v, dst_v,
                  rows_v, gsem, shared, w, wh * HTROWS, HCROWS)


_agg16_call = pl.kernel(
    _agg16_body,
    out_type=jax.ShapeDtypeStruct((NC, NNP, D1), f32),
    mesh=_mesh,
    compiler_params=_sc_params,
    scratch_types=[
        pltpu.VMEM((HCROWS, B), jnp.int32),
        pltpu.VMEM((HCROWS, B), jnp.int32),
        pltpu.VMEM((KB, B, D1), f32),
        pltpu.SemaphoreType.DMA,
        pltpu.VMEM_SHARED((NNP, D1), f32),
    ],
)


def _agg128_body(*refs):
    src_r, dst_r = refs[0], refs[1]
    tabs = refs[2:2 + NCH]
    zeros_h = refs[2 + NCH]
    outs = refs[3 + NCH:3 + 2 * NCH]
    src_v, dst_v, rows_v, gsem, shared = refs[3 + 2 * NCH:]
    w = lax.axis_index("s")
    cid = lax.axis_index("c")
    for c in range(NCH):
        @pl.when(cid == c // (NCH // 2))
        def _(c=c):
            _run_agg_pass(tabs[c], outs[c], src_r, dst_r, zeros_h, src_v,
                          dst_v, rows_v, gsem, shared, w, w * TROWS, CROWS)


_agg128_call = pl.kernel(
    _agg128_body,
    out_type=tuple(jax.ShapeDtypeStruct((NNP, CC), f32) for _ in range(NCH)),
    mesh=_mesh,
    compiler_params=_sc_params,
    scratch_types=[
        pltpu.VMEM((CROWS, B), jnp.int32),
        pltpu.VMEM((CROWS, B), jnp.int32),
        pltpu.VMEM((KB, B, CC), f32),
        pltpu.SemaphoreType.DMA,
        pltpu.VMEM_SHARED((NNP, CC), f32),
    ],
)


def _prep_body(d0_ref, d1_ref, x_ref, dinv_ref, xt_ref):
    dinv = lax.rsqrt(d0_ref[...] + d1_ref[...] + 1.0)
    dinv_ref[...] = dinv
    xt_ref[...] = x_ref[...] * dinv


_prep_call = pl.pallas_call(
    _prep_body,
    grid=(GN,),
    in_specs=[
        pl.BlockSpec((BN, 1), lambda i: (i, 0)),
        pl.BlockSpec((BN, 1), lambda i: (i, 0)),
        pl.BlockSpec((BN, D1), lambda i: (i, 0)),
    ],
    out_specs=[
        pl.BlockSpec((BN, 1), lambda i: (i, 0)),
        pl.BlockSpec((BN, D1), lambda i: (i, 0)),
    ],
    out_shape=[
        jax.ShapeDtypeStruct((NN, 1), f32),
        jax.ShapeDtypeStruct((NN, D1), f32),
    ],
)


def _l1_body(a0_ref, a1_ref, x_ref, dinv_ref, w_ref, b_ref, *outs):
    dinv = dinv_ref[...]
    agg = dinv * (a0_ref[...] + a1_ref[...] + dinv * x_ref[...])
    h = jnp.maximum(_dot(agg, w_ref[...]) + b_ref[...], 0.0)
    ht = dinv * h
    for i, o in enumerate(outs):
        o[...] = ht[:, i * CC:(i + 1) * CC]


_l1_call = pl.pallas_call(
    _l1_body,
    grid=(GN,),
    in_specs=[
        pl.BlockSpec((BN, D1), lambda i: (i, 0)),
        pl.BlockSpec((BN, D1), lambda i: (i, 0)),
        pl.BlockSpec((BN, D1), lambda i: (i, 0)),
        pl.BlockSpec((BN, 1), lambda i: (i, 0)),
        pl.BlockSpec((D1, HID), lambda i: (0, 0)),
        pl.BlockSpec((1, HID), lambda i: (0, 0)),
    ],
    out_specs=[pl.BlockSpec((BN, CC), lambda i: (i, 0)) for _ in range(NCH)],
    out_shape=[jax.ShapeDtypeStruct((NN, CC), f32) for _ in range(NCH)],
)


def _l2_body(*refs):
    ags = refs[:NCH]
    hts = refs[NCH:2 * NCH]
    dinv_ref = refs[2 * NCH]
    w2s = refs[2 * NCH + 1:3 * NCH + 1]
    (b2, wp, bp, ang, wa1, ba1, wa2, ba2,
     wc1g, wc1a, bc1, wc2, bc2, out_ref, acc) = refs[3 * NCH + 1:]
    i = pl.program_id(0)
    dinv = dinv_ref[...]
    t = b2[...]
    for c in range(NCH):
        part = dinv * (ags[c][...] + hts[c][...])
        t = t + _dot(part, w2s[c][...])
    t = jnp.maximum(t, 0.0)
    psum = jnp.sum(t, axis=0, keepdims=True)

    @pl.when(i == 0)
    def _():
        acc[...] = psum

    @pl.when(i > 0)
    def _():
        acc[...] = acc[...] + psum

    @pl.when(i == GN - 1)
    def _():
        g = acc[...] * (1.0 / NN)
        gp = _dot(g, wp[...]) + bp[...]
        a = jnp.maximum(_dot(ang[...], wa1[...]) + ba1[...], 0.0)
        a = jnp.maximum(_dot(a, wa2[...]) + ba2[...], 0.0)
        o = jnp.maximum(_dot(gp, wc1g[...]) + _dot(a, wc1a[...]) + bc1[...],
                        0.0)
        out_ref[...] = _dot(o, wc2[...]) + bc2[...]


def _full(shape):
    return pl.BlockSpec(shape, lambda i: tuple(0 for _ in shape))


_l2_call = pl.pallas_call(
    _l2_body,
    grid=(GN,),
    in_specs=(
        [pl.BlockSpec((BN, CC), lambda i: (i, 0)) for _ in range(2 * NCH)]
        + [pl.BlockSpec((BN, 1), lambda i: (i, 0))]
        + [_full((CC, HID)) for _ in range(NCH)]
        + [_full((1, HID)), _full((HID, HID)), _full((1, HID)),
           _full((1, 12)), _full((12, 32)), _full((1, 32)),
           _full((32, 32)), _full((1, 32)),
           _full((HID, HID)), _full((32, HID)), _full((1, HID)),
           _full((HID, 10)), _full((1, 10))]
    ),
    out_specs=pl.BlockSpec((1, 10), lambda i: (0, 0)),
    out_shape=jax.ShapeDtypeStruct((1, 10), f32),
    scratch_shapes=[pltpu.VMEM((1, HID), f32)],
)


def kernel(x, edge_index, angles, W1, b1, W2, b2, Wp, bp, Wa1, ba1, Wa2, ba2,
           Wc1, bc1, Wc2, bc2):
    src_i = edge_index[0].astype(jnp.int32)
    dst_i = edge_index[1].astype(jnp.int32)
    pad = NE_P - NE
    src_r = jnp.concatenate(
        [src_i, jnp.zeros((pad,), jnp.int32)]).reshape(ROWS, B)
    dst_r = jnp.concatenate(
        [dst_i, jnp.full((pad,), NN, jnp.int32)]).reshape(ROWS, B)
    zeros1 = jnp.zeros((STRIPE,), f32)
    zeros16 = jnp.zeros((STRIPE, CC), f32)
    x16 = jnp.pad(x, ((0, 0), (0, D1 - NODE_IN)))
    w1p = jnp.pad(W1, ((0, D1 - NODE_IN), (0, 0)))

    deg2 = _deg_call(dst_r, zeros1).reshape(NC, NNP, 1)
    dinv, xt = _prep_call(deg2[0], deg2[1], x16)

    agg1 = _agg16_call(src_r, dst_r, xt, zeros16)
    hts = _l1_call(agg1[0], agg1[1], x16, dinv, w1p, b1.reshape(1, HID))

    ags = _agg128_call(src_r, dst_r, *hts, zeros16)

    w2s = [W2[c * CC:(c + 1) * CC, :] for c in range(NCH)]
    out = _l2_call(
        *ags, *hts, dinv, *w2s, b2.reshape(1, HID),
        Wp, bp.reshape(1, HID), angles, Wa1, ba1.reshape(1, 32),
        Wa2, ba2.reshape(1, 32), Wc1[:HID, :], Wc1[HID:, :],
        bc1.reshape(1, HID), Wc2, bc2.reshape(1, 10),
    )
    return out


# speedup vs baseline: 13.5269x; 1.0468x over previous
import jax
import jax.numpy as jnp
from jax import lax
from jax.experimental import pallas as pl
from jax.experimental.pallas import tpu as pltpu
from jax.experimental.pallas import tpu_sc as plsc

f32 = jnp.float32

NN = 50000
NE = 800000
NODE_IN = 4
D1 = 16
HID = 128
CC = 16
NCH = HID // CC

NC, NS = 2, 16
NNP = 51200
STRIPE = NNP // NS
B = 640
ROWS = 1280
NE_P = ROWS * B
TROWS = ROWS // NS
CH = 5
CROWS = TROWS // CH
KB = 4
NB = CROWS // KB
HTROWS = TROWS // 2
HCROWS = HTROWS // CH

BN = 2000
GN = NN // BN

_mesh = plsc.VectorSubcoreMesh(core_axis_name="c", subcore_axis_name="s")
_sc_params = pltpu.CompilerParams(use_tc_tiling_on_sc=False)


def _dot(a, b):
    return lax.dot_general(a, b, (((1,), (0,)), ((), ())),
                           preferred_element_type=f32)


def _deg_body(dst_r, zeros_h, deg_out, dst_v, ones_v, shared):
    w = lax.axis_index("s")
    cid = lax.axis_index("c")
    for i in range(B // 16):
        ones_v[pl.ds(i * 16, 16)] = jnp.ones((16,), f32)
    pltpu.sync_copy(zeros_h, shared.at[pl.ds(w * STRIPE, STRIPE)])
    plsc.subcore_barrier()

    def chunk(c, carry):
        row0 = (cid * NS + w) * HTROWS + c * HCROWS
        pltpu.sync_copy(dst_r.at[pl.ds(row0, HCROWS)], dst_v)

        def blk(j, carry2):
            pltpu.sync_copy(ones_v, shared.at[dst_v.at[j]], add=True)
            return carry2

        return lax.fori_loop(0, HCROWS, blk, carry)

    lax.fori_loop(0, CH, chunk, 0)
    plsc.subcore_barrier()
    out = deg_out.at[cid]
    pltpu.sync_copy(shared.at[pl.ds(w * STRIPE, STRIPE)],
                    out.at[pl.ds(w * STRIPE, STRIPE)])


_deg_call = pl.kernel(
    _deg_body,
    out_type=jax.ShapeDtypeStruct((NC, NNP), f32),
    mesh=_mesh,
    compiler_params=_sc_params,
    scratch_types=[
        pltpu.VMEM((HCROWS, B), jnp.int32),
        pltpu.VMEM((B,), f32),
        pltpu.VMEM_SHARED((NNP,), f32),
    ],
)


def _run_agg_pass(tab, out, src_r, dst_r, zeros_h, src_v, dst_v, rows_v,
                  gsem, shared, w, row_base, crows):
    pltpu.sync_copy(zeros_h, shared.at[pl.ds(w * STRIPE, STRIPE), :])
    plsc.subcore_barrier()
    nb = crows // KB

    def chunk(c, carry):
        row0 = row_base + c * crows
        pltpu.sync_copy(src_r.at[pl.ds(row0, crows)], src_v)
        pltpu.sync_copy(dst_r.at[pl.ds(row0, crows)], dst_v)

        def batch(bb, carry2):
            r0 = bb * KB
            cps = [pltpu.async_copy(tab.at[src_v.at[r0 + k]], rows_v.at[k],
                                    gsem) for k in range(KB)]
            for cp in cps:
                cp.wait()
            for k in range(KB):
                pltpu.sync_copy(rows_v.at[k], shared.at[dst_v.at[r0 + k]],
                                add=True)
            return carry2

        return lax.fori_loop(0, nb, batch, carry)

    lax.fori_loop(0, CH, chunk, 0)
    plsc.subcore_barrier()
    pltpu.sync_copy(shared.at[pl.ds(w * STRIPE, STRIPE), :],
                    out.at[pl.ds(w * STRIPE, STRIPE), :])


def _agg16_body(src_r, dst_r, tab, zeros_h, out, src_v, dst_v, rows_v, gsem,
                shared):
    w = lax.axis_index("s")
    cid = lax.axis_index("c")
    wh = cid * NS + w
    _run_agg_pass(tab, out.at[cid], src_r, dst_r, zeros_h, src_v, dst_v,
                  rows_v, gsem, shared, w, wh * HTROWS, HCROWS)


_agg16_call = pl.kernel(
    _agg16_body,
    out_type=jax.ShapeDtypeStruct((NC, NNP, D1), f32),
    mesh=_mesh,
    compiler_params=_sc_params,
    scratch_types=[
        pltpu.VMEM((HCROWS, B), jnp.int32),
        pltpu.VMEM((HCROWS, B), jnp.int32),
        pltpu.VMEM((KB, B, D1), f32),
        pltpu.SemaphoreType.DMA,
        pltpu.VMEM_SHARED((NNP, D1), f32),
    ],
)


def _agg128_body(*refs):
    src_r, dst_r = refs[0], refs[1]
    tabs = refs[2:2 + NCH]
    zeros_h = refs[2 + NCH]
    outs = refs[3 + NCH:3 + 2 * NCH]
    src_v, dst_v, rows_v, gsem, shared = refs[3 + 2 * NCH:]
    w = lax.axis_index("s")
    cid = lax.axis_index("c")
    for c in range(NCH):
        @pl.when(cid == c // (NCH // 2))
        def _(c=c):
            _run_agg_pass(tabs[c], outs[c], src_r, dst_r, zeros_h, src_v,
                          dst_v, rows_v, gsem, shared, w, w * TROWS, CROWS)


_agg128_call = pl.kernel(
    _agg128_body,
    out_type=tuple(jax.ShapeDtypeStruct((NNP, CC), f32) for _ in range(NCH)),
    mesh=_mesh,
    compiler_params=_sc_params,
    scratch_types=[
        pltpu.VMEM((CROWS, B), jnp.int32),
        pltpu.VMEM((CROWS, B), jnp.int32),
        pltpu.VMEM((KB, B, CC), f32),
        pltpu.SemaphoreType.DMA,
        pltpu.VMEM_SHARED((NNP, CC), f32),
    ],
)


def _prep_body(d0_ref, d1_ref, x_ref, dinv_ref, xt_ref):
    dinv = lax.rsqrt(d0_ref[...] + d1_ref[...] + 1.0)
    dinv_ref[...] = dinv
    xt_ref[...] = x_ref[...] * dinv


_prep_call = pl.pallas_call(
    _prep_body,
    grid=(GN,),
    in_specs=[
        pl.BlockSpec((BN, 1), lambda i: (i, 0)),
        pl.BlockSpec((BN, 1), lambda i: (i, 0)),
        pl.BlockSpec((BN, D1), lambda i: (i, 0)),
    ],
    out_specs=[
        pl.BlockSpec((BN, 1), lambda i: (i, 0)),
        pl.BlockSpec((BN, D1), lambda i: (i, 0)),
    ],
    out_shape=[
        jax.ShapeDtypeStruct((NN, 1), f32),
        jax.ShapeDtypeStruct((NN, D1), f32),
    ],
)


def _l1_body(a0_ref, a1_ref, x_ref, dinv_ref, w_ref, b_ref, *outs):
    dinv = dinv_ref[...]
    agg = dinv * (a0_ref[...] + a1_ref[...] + dinv * x_ref[...])
    h = jnp.maximum(_dot(agg, w_ref[...]) + b_ref[...], 0.0)
    ht = dinv * h
    for i, o in enumerate(outs):
        o[...] = ht[:, i * CC:(i + 1) * CC]


_l1_call = pl.pallas_call(
    _l1_body,
    grid=(GN,),
    in_specs=[
        pl.BlockSpec((BN, D1), lambda i: (i, 0)),
        pl.BlockSpec((BN, D1), lambda i: (i, 0)),
        pl.BlockSpec((BN, D1), lambda i: (i, 0)),
        pl.BlockSpec((BN, 1), lambda i: (i, 0)),
        pl.BlockSpec((D1, HID), lambda i: (0, 0)),
        pl.BlockSpec((1, HID), lambda i: (0, 0)),
    ],
    out_specs=[pl.BlockSpec((BN, CC), lambda i: (i, 0)) for _ in range(NCH)],
    out_shape=[jax.ShapeDtypeStruct((NN, CC), f32) for _ in range(NCH)],
)


def _l2_body(*refs):
    ags = refs[:NCH]
    hts = refs[NCH:2 * NCH]
    dinv_ref = refs[2 * NCH]
    w2s = refs[2 * NCH + 1:3 * NCH + 1]
    (b2, wp, bp, ang, wa1, ba1, wa2, ba2,
     wc1g, wc1a, bc1, wc2, bc2, out_ref, acc) = refs[3 * NCH + 1:]
    i = pl.program_id(0)
    dinv = dinv_ref[...]
    t = b2[...]
    for c in range(NCH):
        part = dinv * (ags[c][...] + hts[c][...])
        t = t + _dot(part, w2s[c][...])
    t = jnp.maximum(t, 0.0)
    psum = jnp.sum(t, axis=0, keepdims=True)

    @pl.when(i == 0)
    def _():
        acc[...] = psum

    @pl.when(i > 0)
    def _():
        acc[...] = acc[...] + psum

    @pl.when(i == GN - 1)
    def _():
        g = acc[...] * (1.0 / NN)
        gp = _dot(g, wp[...]) + bp[...]
        a = jnp.maximum(_dot(ang[...], wa1[...]) + ba1[...], 0.0)
        a = jnp.maximum(_dot(a, wa2[...]) + ba2[...], 0.0)
        o = jnp.maximum(_dot(gp, wc1g[...]) + _dot(a, wc1a[...]) + bc1[...],
                        0.0)
        out_ref[...] = _dot(o, wc2[...]) + bc2[...]


def _full(shape):
    return pl.BlockSpec(shape, lambda i: tuple(0 for _ in shape))


_l2_call = pl.pallas_call(
    _l2_body,
    grid=(GN,),
    in_specs=(
        [pl.BlockSpec((BN, CC), lambda i: (i, 0)) for _ in range(2 * NCH)]
        + [pl.BlockSpec((BN, 1), lambda i: (i, 0))]
        + [_full((CC, HID)) for _ in range(NCH)]
        + [_full((1, HID)), _full((HID, HID)), _full((1, HID)),
           _full((1, 12)), _full((12, 32)), _full((1, 32)),
           _full((32, 32)), _full((1, 32)),
           _full((HID, HID)), _full((32, HID)), _full((1, HID)),
           _full((HID, 10)), _full((1, 10))]
    ),
    out_specs=pl.BlockSpec((1, 10), lambda i: (0, 0)),
    out_shape=jax.ShapeDtypeStruct((1, 10), f32),
    scratch_shapes=[pltpu.VMEM((1, HID), f32)],
)


def kernel(x, edge_index, angles, W1, b1, W2, b2, Wp, bp, Wa1, ba1, Wa2, ba2,
           Wc1, bc1, Wc2, bc2):
    src_i = edge_index[0].astype(jnp.int32)
    dst_i = edge_index[1].astype(jnp.int32)
    pad = NE_P - NE
    src_r = jnp.concatenate(
        [src_i, jnp.zeros((pad,), jnp.int32)]).reshape(ROWS, B)
    pad_dst = NN + jnp.arange(pad, dtype=jnp.int32) % (NNP - NN)
    dst_r = jnp.concatenate([dst_i, pad_dst]).reshape(ROWS, B)
    zeros1 = jnp.zeros((STRIPE,), f32)
    zeros16 = jnp.zeros((STRIPE, CC), f32)
    x16 = jnp.pad(x, ((0, 0), (0, D1 - NODE_IN)))
    w1p = jnp.pad(W1, ((0, D1 - NODE_IN), (0, 0)))

    deg2 = _deg_call(dst_r, zeros1).reshape(NC, NNP, 1)
    dinv, xt = _prep_call(deg2[0], deg2[1], x16)

    agg1 = _agg16_call(src_r, dst_r, xt, zeros16)
    hts = _l1_call(agg1[0], agg1[1], x16, dinv, w1p, b1.reshape(1, HID))

    ags = _agg128_call(src_r, dst_r, *hts, zeros16)

    w2s = [W2[c * CC:(c + 1) * CC, :] for c in range(NCH)]
    out = _l2_call(
        *ags, *hts, dinv, *w2s, b2.reshape(1, HID),
        Wp, bp.reshape(1, HID), angles, Wa1, ba1.reshape(1, 32),
        Wa2, ba2.reshape(1, 32), Wc1[:HID, :], Wc1[HID:, :],
        bc1.reshape(1, HID), Wc2, bc2.reshape(1, 10),
    )
    return out


# speedup vs baseline: 14.6521x; 1.0832x over previous
import jax
import jax.numpy as jnp
from jax import lax
from jax.experimental import pallas as pl
from jax.experimental.pallas import tpu as pltpu
from jax.experimental.pallas import tpu_sc as plsc

f32 = jnp.float32

NN = 50000
NE = 800000
NODE_IN = 4
D1 = 16
HID = 128
CC = 16
NCH = HID // CC

NC, NS = 2, 16
NNP = 51200
STRIPE = NNP // NS
B = 640
ROWS = 1280
NE_P = ROWS * B
TROWS = ROWS // NS
CH = 5
CROWS = TROWS // CH
KB = 2
NB = CROWS // KB
HTROWS = TROWS // 2
HCROWS = HTROWS // CH

BN = 2000
GN = NN // BN

_mesh = plsc.VectorSubcoreMesh(core_axis_name="c", subcore_axis_name="s")
_sc_params = pltpu.CompilerParams(use_tc_tiling_on_sc=False)


def _dot(a, b):
    return lax.dot_general(a, b, (((1,), (0,)), ((), ())),
                           preferred_element_type=f32)


def _deg_body(dst_r, zeros_h, deg_out, dst_v, ones_v, shared):
    w = lax.axis_index("s")
    cid = lax.axis_index("c")
    for i in range(B // 16):
        ones_v[pl.ds(i * 16, 16)] = jnp.ones((16,), f32)
    pltpu.sync_copy(zeros_h, shared.at[pl.ds(w * STRIPE, STRIPE)])
    plsc.subcore_barrier()

    def chunk(c, carry):
        row0 = (cid * NS + w) * HTROWS + c * HCROWS
        pltpu.sync_copy(dst_r.at[pl.ds(row0, HCROWS)], dst_v)

        def blk(j, carry2):
            pltpu.sync_copy(ones_v, shared.at[dst_v.at[j]], add=True)
            return carry2

        return lax.fori_loop(0, HCROWS, blk, carry)

    lax.fori_loop(0, CH, chunk, 0)
    plsc.subcore_barrier()
    out = deg_out.at[cid]
    pltpu.sync_copy(shared.at[pl.ds(w * STRIPE, STRIPE)],
                    out.at[pl.ds(w * STRIPE, STRIPE)])


_deg_call = pl.kernel(
    _deg_body,
    out_type=jax.ShapeDtypeStruct((NC, NNP), f32),
    mesh=_mesh,
    compiler_params=_sc_params,
    scratch_types=[
        pltpu.VMEM((HCROWS, B), jnp.int32),
        pltpu.VMEM((B,), f32),
        pltpu.VMEM_SHARED((NNP,), f32),
    ],
)


def _run_agg_pass(tab, out, src_r, dst_r, zeros_h, src_v, dst_v, rows_v,
                  gsems, ssems, shared, w, row_base, crows):
    pltpu.sync_copy(zeros_h, shared.at[pl.ds(w * STRIPE, STRIPE), :])
    plsc.subcore_barrier()
    nb = crows // KB

    def fire_gather(tab_, bb):
        buf = bb % 2
        return [pltpu.async_copy(tab_.at[src_v.at[bb * KB + k]],
                                 rows_v.at[buf, k], gsems[buf])
                for k in range(KB)]

    def fire_scatter(bb):
        buf = bb % 2
        return [pltpu.async_copy(rows_v.at[buf, k],
                                 shared.at[dst_v.at[bb * KB + k]],
                                 ssems[buf], add=True)
                for k in range(KB)]

    def chunk(c, carry):
        row0 = row_base + c * crows
        pltpu.sync_copy(src_r.at[pl.ds(row0, crows)], src_v)
        pltpu.sync_copy(dst_r.at[pl.ds(row0, crows)], dst_v)
        gcp = [None] * nb
        scp = [None] * nb
        gcp[0] = fire_gather(tab, 0)
        for bb in range(nb):
            if bb + 1 < nb:
                if bb >= 1:
                    for cp in scp[bb - 1]:
                        cp.wait()
                gcp[bb + 1] = fire_gather(tab, bb + 1)
            for cp in gcp[bb]:
                cp.wait()
            scp[bb] = fire_scatter(bb)
        for bb in range(max(nb - 2, 0), nb):
            for cp in scp[bb]:
                cp.wait()
        return carry

    lax.fori_loop(0, CH, chunk, 0)
    plsc.subcore_barrier()
    pltpu.sync_copy(shared.at[pl.ds(w * STRIPE, STRIPE), :],
                    out.at[pl.ds(w * STRIPE, STRIPE), :])


def _agg16_body(src_r, dst_r, tab, zeros_h, out, src_v, dst_v, rows_v,
                g0, g1, s0, s1, shared):
    w = lax.axis_index("s")
    cid = lax.axis_index("c")
    wh = cid * NS + w
    _run_agg_pass(tab, out.at[cid], src_r, dst_r, zeros_h, src_v, dst_v,
                  rows_v, (g0, g1), (s0, s1), shared, w, wh * HTROWS, HCROWS)


_agg16_call = pl.kernel(
    _agg16_body,
    out_type=jax.ShapeDtypeStruct((NC, NNP, D1), f32),
    mesh=_mesh,
    compiler_params=_sc_params,
    scratch_types=[
        pltpu.VMEM((HCROWS, B), jnp.int32),
        pltpu.VMEM((HCROWS, B), jnp.int32),
        pltpu.VMEM((2, KB, B, D1), f32),
        pltpu.SemaphoreType.DMA,
        pltpu.SemaphoreType.DMA,
        pltpu.SemaphoreType.DMA,
        pltpu.SemaphoreType.DMA,
        pltpu.VMEM_SHARED((NNP, D1), f32),
    ],
)


def _agg128_body(*refs):
    src_r, dst_r = refs[0], refs[1]
    tabs = refs[2:2 + NCH]
    zeros_h = refs[2 + NCH]
    outs = refs[3 + NCH:3 + 2 * NCH]
    src_v, dst_v, rows_v, g0, g1, s0, s1, shared = refs[3 + 2 * NCH:]
    w = lax.axis_index("s")
    cid = lax.axis_index("c")
    for c in range(NCH):
        @pl.when(cid == c // (NCH // 2))
        def _(c=c):
            _run_agg_pass(tabs[c], outs[c], src_r, dst_r, zeros_h, src_v,
                          dst_v, rows_v, (g0, g1), (s0, s1), shared, w,
                          w * TROWS, CROWS)


_agg128_call = pl.kernel(
    _agg128_body,
    out_type=tuple(jax.ShapeDtypeStruct((NNP, CC), f32) for _ in range(NCH)),
    mesh=_mesh,
    compiler_params=_sc_params,
    scratch_types=[
        pltpu.VMEM((CROWS, B), jnp.int32),
        pltpu.VMEM((CROWS, B), jnp.int32),
        pltpu.VMEM((2, KB, B, CC), f32),
        pltpu.SemaphoreType.DMA,
        pltpu.SemaphoreType.DMA,
        pltpu.SemaphoreType.DMA,
        pltpu.SemaphoreType.DMA,
        pltpu.VMEM_SHARED((NNP, CC), f32),
    ],
)


def _prep_body(d0_ref, d1_ref, x_ref, dinv_ref, xt_ref):
    dinv = lax.rsqrt(d0_ref[...] + d1_ref[...] + 1.0)
    dinv_ref[...] = dinv
    xt_ref[...] = x_ref[...] * dinv


_prep_call = pl.pallas_call(
    _prep_body,
    grid=(GN,),
    in_specs=[
        pl.BlockSpec((BN, 1), lambda i: (i, 0)),
        pl.BlockSpec((BN, 1), lambda i: (i, 0)),
        pl.BlockSpec((BN, D1), lambda i: (i, 0)),
    ],
    out_specs=[
        pl.BlockSpec((BN, 1), lambda i: (i, 0)),
        pl.BlockSpec((BN, D1), lambda i: (i, 0)),
    ],
    out_shape=[
        jax.ShapeDtypeStruct((NN, 1), f32),
        jax.ShapeDtypeStruct((NN, D1), f32),
    ],
)


def _l1_body(a0_ref, a1_ref, x_ref, dinv_ref, w_ref, b_ref, *outs):
    dinv = dinv_ref[...]
    agg = dinv * (a0_ref[...] + a1_ref[...] + dinv * x_ref[...])
    h = jnp.maximum(_dot(agg, w_ref[...]) + b_ref[...], 0.0)
    ht = dinv * h
    for i, o in enumerate(outs):
        o[...] = ht[:, i * CC:(i + 1) * CC]


_l1_call = pl.pallas_call(
    _l1_body,
    grid=(GN,),
    in_specs=[
        pl.BlockSpec((BN, D1), lambda i: (i, 0)),
        pl.BlockSpec((BN, D1), lambda i: (i, 0)),
        pl.BlockSpec((BN, D1), lambda i: (i, 0)),
        pl.BlockSpec((BN, 1), lambda i: (i, 0)),
        pl.BlockSpec((D1, HID), lambda i: (0, 0)),
        pl.BlockSpec((1, HID), lambda i: (0, 0)),
    ],
    out_specs=[pl.BlockSpec((BN, CC), lambda i: (i, 0)) for _ in range(NCH)],
    out_shape=[jax.ShapeDtypeStruct((NN, CC), f32) for _ in range(NCH)],
)


def _l2_body(*refs):
    ags = refs[:NCH]
    hts = refs[NCH:2 * NCH]
    dinv_ref = refs[2 * NCH]
    w2s = refs[2 * NCH + 1:3 * NCH + 1]
    (b2, wp, bp, ang, wa1, ba1, wa2, ba2,
     wc1g, wc1a, bc1, wc2, bc2, out_ref, acc) = refs[3 * NCH + 1:]
    i = pl.program_id(0)
    dinv = dinv_ref[...]
    t = b2[...]
    for c in range(NCH):
        part = dinv * (ags[c][...] + hts[c][...])
        t = t + _dot(part, w2s[c][...])
    t = jnp.maximum(t, 0.0)
    psum = jnp.sum(t, axis=0, keepdims=True)

    @pl.when(i == 0)
    def _():
        acc[...] = psum

    @pl.when(i > 0)
    def _():
        acc[...] = acc[...] + psum

    @pl.when(i == GN - 1)
    def _():
        g = acc[...] * (1.0 / NN)
        gp = _dot(g, wp[...]) + bp[...]
        a = jnp.maximum(_dot(ang[...], wa1[...]) + ba1[...], 0.0)
        a = jnp.maximum(_dot(a, wa2[...]) + ba2[...], 0.0)
        o = jnp.maximum(_dot(gp, wc1g[...]) + _dot(a, wc1a[...]) + bc1[...],
                        0.0)
        out_ref[...] = _dot(o, wc2[...]) + bc2[...]


def _full(shape):
    return pl.BlockSpec(shape, lambda i: tuple(0 for _ in shape))


_l2_call = pl.pallas_call(
    _l2_body,
    grid=(GN,),
    in_specs=(
        [pl.BlockSpec((BN, CC), lambda i: (i, 0)) for _ in range(2 * NCH)]
        + [pl.BlockSpec((BN, 1), lambda i: (i, 0))]
        + [_full((CC, HID)) for _ in range(NCH)]
        + [_full((1, HID)), _full((HID, HID)), _full((1, HID)),
           _full((1, 12)), _full((12, 32)), _full((1, 32)),
           _full((32, 32)), _full((1, 32)),
           _full((HID, HID)), _full((32, HID)), _full((1, HID)),
           _full((HID, 10)), _full((1, 10))]
    ),
    out_specs=pl.BlockSpec((1, 10), lambda i: (0, 0)),
    out_shape=jax.ShapeDtypeStruct((1, 10), f32),
    scratch_shapes=[pltpu.VMEM((1, HID), f32)],
)


def kernel(x, edge_index, angles, W1, b1, W2, b2, Wp, bp, Wa1, ba1, Wa2, ba2,
           Wc1, bc1, Wc2, bc2):
    src_i = edge_index[0].astype(jnp.int32)
    dst_i = edge_index[1].astype(jnp.int32)
    pad = NE_P - NE
    src_r = jnp.concatenate(
        [src_i, jnp.zeros((pad,), jnp.int32)]).reshape(ROWS, B)
    pad_dst = NN + jnp.arange(pad, dtype=jnp.int32) % (NNP - NN)
    dst_r = jnp.concatenate([dst_i, pad_dst]).reshape(ROWS, B)
    zeros1 = jnp.zeros((STRIPE,), f32)
    zeros16 = jnp.zeros((STRIPE, CC), f32)
    x16 = jnp.pad(x, ((0, 0), (0, D1 - NODE_IN)))
    w1p = jnp.pad(W1, ((0, D1 - NODE_IN), (0, 0)))

    deg2 = _deg_call(dst_r, zeros1).reshape(NC, NNP, 1)
    dinv, xt = _prep_call(deg2[0], deg2[1], x16)

    agg1 = _agg16_call(src_r, dst_r, xt, zeros16)
    hts = _l1_call(agg1[0], agg1[1], x16, dinv, w1p, b1.reshape(1, HID))

    ags = _agg128_call(src_r, dst_r, *hts, zeros16)

    w2s = [W2[c * CC:(c + 1) * CC, :] for c in range(NCH)]
    out = _l2_call(
        *ags, *hts, dinv, *w2s, b2.reshape(1, HID),
        Wp, bp.reshape(1, HID), angles, Wa1, ba1.reshape(1, 32),
        Wa2, ba2.reshape(1, 32), Wc1[:HID, :], Wc1[HID:, :],
        bc1.reshape(1, HID), Wc2, bc2.reshape(1, 10),
    )
    return out


# speedup vs baseline: 21.9829x; 1.5003x over previous
import jax
import jax.numpy as jnp
from jax import lax
from jax.experimental import pallas as pl
from jax.experimental.pallas import tpu as pltpu
from jax.experimental.pallas import tpu_sc as plsc

f32 = jnp.float32

NN = 50000
NE = 800000
NODE_IN = 4
D1 = 16
HID = 128
CC = 16
NCH = HID // CC

NC, NS = 2, 16
NNP = 51200
STRIPE = NNP // NS
B = 640
ROWS = 1280
NE_P = ROWS * B
TROWS = ROWS // NS
CH = 5
CROWS = TROWS // CH
KB = 2
NB = CROWS // KB
HTROWS = TROWS // 2
HCROWS = HTROWS // CH

BN = 2000
GN = NN // BN

_mesh = plsc.VectorSubcoreMesh(core_axis_name="c", subcore_axis_name="s")
_sc_params = pltpu.CompilerParams(use_tc_tiling_on_sc=False)


def _dot(a, b):
    return lax.dot_general(a, b, (((1,), (0,)), ((), ())),
                           preferred_element_type=f32)


def _deg_body(dst_r, zeros_h, deg_out, dst_v, ones_v, shared):
    w = lax.axis_index("s")
    cid = lax.axis_index("c")
    for i in range(B // 16):
        ones_v[pl.ds(i * 16, 16)] = jnp.ones((16,), f32)
    pltpu.sync_copy(zeros_h, shared.at[pl.ds(w * STRIPE, STRIPE)])
    plsc.subcore_barrier()

    def chunk(c, carry):
        row0 = (cid * NS + w) * HTROWS + c * HCROWS
        pltpu.sync_copy(dst_r.at[pl.ds(row0, HCROWS)], dst_v)

        def blk(j, carry2):
            pltpu.sync_copy(ones_v, shared.at[dst_v.at[j]], add=True)
            return carry2

        return lax.fori_loop(0, HCROWS, blk, carry)

    lax.fori_loop(0, CH, chunk, 0)
    plsc.subcore_barrier()
    out = deg_out.at[cid]
    pltpu.sync_copy(shared.at[pl.ds(w * STRIPE, STRIPE)],
                    out.at[pl.ds(w * STRIPE, STRIPE)])


_deg_call = pl.kernel(
    _deg_body,
    out_type=jax.ShapeDtypeStruct((NC, NNP), f32),
    mesh=_mesh,
    compiler_params=_sc_params,
    scratch_types=[
        pltpu.VMEM((HCROWS, B), jnp.int32),
        pltpu.VMEM((B,), f32),
        pltpu.VMEM_SHARED((NNP,), f32),
    ],
)


def _run_agg_pass(tab, out, src_r, dst_r, zeros_h, src_v, dst_v, rows_v,
                  gsems, ssems, shared, w, row_base, crows):
    pltpu.sync_copy(zeros_h, shared.at[pl.ds(w * STRIPE, STRIPE), :])
    plsc.subcore_barrier()
    nb = crows // KB

    def fire_gather(tab_, bb):
        buf = bb % 2
        return [pltpu.async_copy(tab_.at[src_v.at[bb * KB + k]],
                                 rows_v.at[buf, k], gsems[buf])
                for k in range(KB)]

    def fire_scatter(bb):
        buf = bb % 2
        return [pltpu.async_copy(rows_v.at[buf, k],
                                 shared.at[dst_v.at[bb * KB + k]],
                                 ssems[buf], add=True)
                for k in range(KB)]

    def chunk(c, carry):
        row0 = row_base + c * crows
        pltpu.sync_copy(src_r.at[pl.ds(row0, crows)], src_v)
        pltpu.sync_copy(dst_r.at[pl.ds(row0, crows)], dst_v)
        gcp = [None] * nb
        scp = [None] * nb
        gcp[0] = fire_gather(tab, 0)
        for bb in range(nb):
            if bb + 1 < nb:
                if bb >= 1:
                    for cp in scp[bb - 1]:
                        cp.wait()
                gcp[bb + 1] = fire_gather(tab, bb + 1)
            for cp in gcp[bb]:
                cp.wait()
            scp[bb] = fire_scatter(bb)
        for bb in range(max(nb - 2, 0), nb):
            for cp in scp[bb]:
                cp.wait()
        return carry

    lax.fori_loop(0, CH, chunk, 0)
    plsc.subcore_barrier()
    pltpu.sync_copy(shared.at[pl.ds(w * STRIPE, STRIPE), :],
                    out.at[pl.ds(w * STRIPE, STRIPE), :])


def _agg16_body(src_r, dst_r, tab, zeros_h, out, src_v, dst_v, rows_v,
                g0, g1, s0, s1, shared):
    w = lax.axis_index("s")
    cid = lax.axis_index("c")
    wh = cid * NS + w
    _run_agg_pass(tab, out.at[cid], src_r, dst_r, zeros_h, src_v, dst_v,
                  rows_v, (g0, g1), (s0, s1), shared, w, wh * HTROWS, HCROWS)


_agg16_call = pl.kernel(
    _agg16_body,
    out_type=jax.ShapeDtypeStruct((NC, NNP, D1), f32),
    mesh=_mesh,
    compiler_params=_sc_params,
    scratch_types=[
        pltpu.VMEM((HCROWS, B), jnp.int32),
        pltpu.VMEM((HCROWS, B), jnp.int32),
        pltpu.VMEM((2, KB, B, D1), f32),
        pltpu.SemaphoreType.DMA,
        pltpu.SemaphoreType.DMA,
        pltpu.SemaphoreType.DMA,
        pltpu.SemaphoreType.DMA,
        pltpu.VMEM_SHARED((NNP, D1), f32),
    ],
)


def _agg128_body(*refs):
    src_r, dst_r = refs[0], refs[1]
    tabs = refs[2:2 + NCH]
    zeros_h = refs[2 + NCH]
    outs = refs[3 + NCH:3 + 2 * NCH]
    src_v, dst_v, rows_v, g0, g1, s0, s1, shared = refs[3 + 2 * NCH:]
    w = lax.axis_index("s")
    cid = lax.axis_index("c")
    for c in range(NCH):
        @pl.when(cid == c // (NCH // 2))
        def _(c=c):
            _run_agg_pass(tabs[c], outs[c], src_r, dst_r, zeros_h, src_v,
                          dst_v, rows_v, (g0, g1), (s0, s1), shared, w,
                          w * TROWS, CROWS)


_agg128_call = pl.kernel(
    _agg128_body,
    out_type=tuple(jax.ShapeDtypeStruct((NNP, CC), f32) for _ in range(NCH)),
    mesh=_mesh,
    compiler_params=_sc_params,
    scratch_types=[
        pltpu.VMEM((CROWS, B), jnp.int32),
        pltpu.VMEM((CROWS, B), jnp.int32),
        pltpu.VMEM((2, KB, B, CC), f32),
        pltpu.SemaphoreType.DMA,
        pltpu.SemaphoreType.DMA,
        pltpu.SemaphoreType.DMA,
        pltpu.SemaphoreType.DMA,
        pltpu.VMEM_SHARED((NNP, CC), f32),
    ],
)


def _prep_body(d0_ref, d1_ref, x_ref, dinv_ref, xt_ref):
    dinv = lax.rsqrt(d0_ref[...] + d1_ref[...] + 1.0)
    dinv_ref[...] = dinv
    xt_ref[...] = x_ref[...] * dinv


_prep_call = pl.pallas_call(
    _prep_body,
    grid=(GN,),
    in_specs=[
        pl.BlockSpec((BN, 1), lambda i: (i, 0)),
        pl.BlockSpec((BN, 1), lambda i: (i, 0)),
        pl.BlockSpec((BN, D1), lambda i: (i, 0)),
    ],
    out_specs=[
        pl.BlockSpec((BN, 1), lambda i: (i, 0)),
        pl.BlockSpec((BN, D1), lambda i: (i, 0)),
    ],
    out_shape=[
        jax.ShapeDtypeStruct((NN, 1), f32),
        jax.ShapeDtypeStruct((NN, D1), f32),
    ],
)


def _l1_body(a0_ref, a1_ref, x_ref, dinv_ref, w_ref, b_ref, *outs):
    dinv = dinv_ref[...]
    agg = dinv * (a0_ref[...] + a1_ref[...] + dinv * x_ref[...])
    h = jnp.maximum(_dot(agg, w_ref[...]) + b_ref[...], 0.0)
    ht = dinv * h
    for i, o in enumerate(outs):
        o[...] = ht[:, i * CC:(i + 1) * CC]


_l1_call = pl.pallas_call(
    _l1_body,
    grid=(GN,),
    in_specs=[
        pl.BlockSpec((BN, D1), lambda i: (i, 0)),
        pl.BlockSpec((BN, D1), lambda i: (i, 0)),
        pl.BlockSpec((BN, D1), lambda i: (i, 0)),
        pl.BlockSpec((BN, 1), lambda i: (i, 0)),
        pl.BlockSpec((D1, HID), lambda i: (0, 0)),
        pl.BlockSpec((1, HID), lambda i: (0, 0)),
    ],
    out_specs=[pl.BlockSpec((BN, CC), lambda i: (i, 0)) for _ in range(NCH)],
    out_shape=[jax.ShapeDtypeStruct((NN, CC), f32) for _ in range(NCH)],
)


def _l2_body(*refs):
    ags = refs[:NCH]
    hts = refs[NCH:2 * NCH]
    dinv_ref = refs[2 * NCH]
    w2s = refs[2 * NCH + 1:3 * NCH + 1]
    (b2, wp, bp, ang, wa1, ba1, wa2, ba2,
     wc1g, wc1a, bc1, wc2, bc2, out_ref, acc) = refs[3 * NCH + 1:]
    i = pl.program_id(0)
    dinv = dinv_ref[...]
    t = b2[...]
    for c in range(NCH):
        part = dinv * (ags[c][...] + hts[c][...])
        t = t + _dot(part, w2s[c][...])
    t = jnp.maximum(t, 0.0)
    psum = jnp.sum(t, axis=0, keepdims=True)

    @pl.when(i == 0)
    def _():
        acc[...] = psum

    @pl.when(i > 0)
    def _():
        acc[...] = acc[...] + psum

    @pl.when(i == GN - 1)
    def _():
        g = acc[...] * (1.0 / NN)
        gp = _dot(g, wp[...]) + bp[...]
        a = jnp.maximum(_dot(ang[...], wa1[...]) + ba1[...], 0.0)
        a = jnp.maximum(_dot(a, wa2[...]) + ba2[...], 0.0)
        o = jnp.maximum(_dot(gp, wc1g[...]) + _dot(a, wc1a[...]) + bc1[...],
                        0.0)
        out_ref[...] = _dot(o, wc2[...]) + bc2[...]


def _full(shape):
    return pl.BlockSpec(shape, lambda i: tuple(0 for _ in shape))


_l2_call = pl.pallas_call(
    _l2_body,
    grid=(GN,),
    in_specs=(
        [pl.BlockSpec((BN, CC), lambda i: (i, 0)) for _ in range(2 * NCH)]
        + [pl.BlockSpec((BN, 1), lambda i: (i, 0))]
        + [_full((CC, HID)) for _ in range(NCH)]
        + [_full((1, HID)), _full((HID, HID)), _full((1, HID)),
           _full((1, 12)), _full((12, 32)), _full((1, 32)),
           _full((32, 32)), _full((1, 32)),
           _full((HID, HID)), _full((32, HID)), _full((1, HID)),
           _full((HID, 10)), _full((1, 10))]
    ),
    out_specs=pl.BlockSpec((1, 10), lambda i: (0, 0)),
    out_shape=jax.ShapeDtypeStruct((1, 10), f32),
    scratch_shapes=[pltpu.VMEM((1, HID), f32)],
)


def kernel(x, edge_index, angles, W1, b1, W2, b2, Wp, bp, Wa1, ba1, Wa2, ba2,
           Wc1, bc1, Wc2, bc2):
    src_i = edge_index[0].astype(jnp.int32)
    dst_i = edge_index[1].astype(jnp.int32)
    pad = NE_P - NE
    pad_idx = jnp.arange(pad, dtype=jnp.int32)
    src_r = jnp.concatenate([src_i, pad_idx % NN]).reshape(ROWS, B)
    dst_r = jnp.concatenate(
        [dst_i, NN + pad_idx % (NNP - NN)]).reshape(ROWS, B)
    zeros1 = jnp.zeros((STRIPE,), f32)
    zeros16 = jnp.zeros((STRIPE, CC), f32)
    x16 = jnp.pad(x, ((0, 0), (0, D1 - NODE_IN)))
    w1p = jnp.pad(W1, ((0, D1 - NODE_IN), (0, 0)))

    deg2 = _deg_call(dst_r, zeros1).reshape(NC, NNP, 1)
    dinv, xt = _prep_call(deg2[0], deg2[1], x16)

    agg1 = _agg16_call(src_r, dst_r, xt, zeros16)
    hts = _l1_call(agg1[0], agg1[1], x16, dinv, w1p, b1.reshape(1, HID))

    ags = _agg128_call(src_r, dst_r, *hts, zeros16)

    w2s = [W2[c * CC:(c + 1) * CC, :] for c in range(NCH)]
    out = _l2_call(
        *ags, *hts, dinv, *w2s, b2.reshape(1, HID),
        Wp, bp.reshape(1, HID), angles, Wa1, ba1.reshape(1, 32),
        Wa2, ba2.reshape(1, 32), Wc1[:HID, :], Wc1[HID:, :],
        bc1.reshape(1, HID), Wc2, bc2.reshape(1, 10),
    )
    return out
